# Initial kernel scaffold; baseline (speedup 1.0000x reference)
import jax
import jax.numpy as jnp
from jax.experimental import pallas as pl


def kernel(x, edge_index, W_conv, b_conv, W1, b1, W2, b2, W3, b3):
    raise NotImplementedError("write your pallas kernel here")


# speedup vs baseline: 20.2517x; 20.2517x over previous
import functools

import jax
import jax.numpy as jnp
from jax import lax
from jax.experimental import pallas as pl
from jax.experimental.pallas import tpu as pltpu
from jax.experimental.pallas import tpu_sc as plsc

N_NODES = 10000
D_FEAT = 128
D_HALF = 64
N_EDGES = 320000

NC, NS, L = 2, 16, 16
NW = NC * NS
E_CHUNK = 128
N_PAD = 10240
ROWS_PER_TILE = N_PAD // NS
CHUNKS = 2560
E_PAD = CHUNKS * E_CHUNK
CH_PER_TILE_DEG = CHUNKS // NW
CH_PER_CORE = CHUNKS // NC
CH_PER_TILE_ACC = CHUNKS // NS

_mesh = plsc.VectorSubcoreMesh(core_axis_name="c", subcore_axis_name="s")


@functools.partial(
    pl.kernel,
    out_type=jax.ShapeDtypeStruct((NC, N_PAD, 16), jnp.float32),
    mesh=_mesh,
    scratch_types=[
        pltpu.VMEM((CH_PER_TILE_DEG, E_CHUNK), jnp.int32),
        pltpu.VMEM((E_CHUNK, 16), jnp.float32),
        pltpu.VMEM((E_CHUNK, 16), jnp.float32),
        pltpu.VMEM_SHARED((N_PAD, 16), jnp.float32),
    ],
)
def _deg_kernel(dst_hbm, deg_out, idx_d, ones_b, zero_b, deg_sh):
    c = lax.axis_index("c")
    s = lax.axis_index("s")

    def fill(i, _):
        ones_b[i, :] = jnp.ones((16,), jnp.float32)
        zero_b[i, :] = jnp.zeros((16,), jnp.float32)
        return _
    lax.fori_loop(0, E_CHUNK, fill, None)

    t0 = s * ROWS_PER_TILE
    for k in range(ROWS_PER_TILE // E_CHUNK):
        pltpu.sync_copy(zero_b, deg_sh.at[pl.ds(t0 + k * E_CHUNK, E_CHUNK)])
    plsc.subcore_barrier()

    base = c * CH_PER_CORE + s * CH_PER_TILE_DEG
    pltpu.sync_copy(dst_hbm.at[pl.ds(base, CH_PER_TILE_DEG)], idx_d)

    def body(j, _):
        pltpu.sync_copy(ones_b, deg_sh.at[idx_d.at[j]], add=True)
        return _
    lax.fori_loop(0, CH_PER_TILE_DEG, body, None)
    plsc.subcore_barrier()

    pltpu.sync_copy(deg_sh.at[pl.ds(t0, ROWS_PER_TILE)],
                    deg_out.at[c, pl.ds(t0, ROWS_PER_TILE)])


@functools.partial(
    pl.kernel,
    out_type=jax.ShapeDtypeStruct((NC, N_PAD, D_HALF), jnp.float32),
    mesh=_mesh,
    scratch_types=[
        pltpu.VMEM((CH_PER_TILE_ACC, E_CHUNK), jnp.int32),
        pltpu.VMEM((CH_PER_TILE_ACC, E_CHUNK), jnp.int32),
        pltpu.VMEM((E_CHUNK, D_HALF), jnp.float32),
        pltpu.VMEM((E_CHUNK, D_HALF), jnp.float32),
        pltpu.VMEM_SHARED((N_PAD, D_HALF), jnp.float32),
        pltpu.SemaphoreType.DMA,
    ],
    compiler_params=pltpu.CompilerParams(use_tc_tiling_on_sc=False),
)
def _scatter_kernel(src_hbm, dst_hbm, ys0_hbm, ys1_hbm, acc_out,
                    idx_s, idx_d, rows, zero_b, acc_sh, sem):
    c = lax.axis_index("c")
    s = lax.axis_index("s")

    def fill(i, _):
        for k in range(D_HALF // L):
            zero_b[i, pl.ds(k * L, L)] = jnp.zeros((L,), jnp.float32)
        return _
    lax.fori_loop(0, E_CHUNK, fill, None)

    t0 = s * ROWS_PER_TILE
    for k in range(ROWS_PER_TILE // E_CHUNK):
        pltpu.sync_copy(zero_b, acc_sh.at[pl.ds(t0 + k * E_CHUNK, E_CHUNK)])
    plsc.subcore_barrier()

    base = s * CH_PER_TILE_ACC
    pltpu.sync_copy(src_hbm.at[pl.ds(base, CH_PER_TILE_ACC)], idx_s)
    pltpu.sync_copy(dst_hbm.at[pl.ds(base, CH_PER_TILE_ACC)], idx_d)

    def body0(j, _):
        pltpu.async_copy(ys0_hbm.at[idx_s.at[j]], rows, sem).wait()
        pltpu.sync_copy(rows, acc_sh.at[idx_d.at[j]], add=True)
        return _

    def body1(j, _):
        pltpu.async_copy(ys1_hbm.at[idx_s.at[j]], rows, sem).wait()
        pltpu.sync_copy(rows, acc_sh.at[idx_d.at[j]], add=True)
        return _

    @pl.when(c == 0)
    def _():
        lax.fori_loop(0, CH_PER_TILE_ACC, body0, None)

    @pl.when(c == 1)
    def _():
        lax.fori_loop(0, CH_PER_TILE_ACC, body1, None)
    plsc.subcore_barrier()

    pltpu.sync_copy(acc_sh.at[pl.ds(t0, ROWS_PER_TILE)],
                    acc_out.at[c, pl.ds(t0, ROWS_PER_TILE)])


def _scale_body(x_ref, w_ref, da_ref, db_ref, ys0_ref, ys1_ref):
    deg = 1.0 + da_ref[:, :1] + db_ref[:, :1]
    dinv = lax.rsqrt(deg)
    xw = jnp.dot(x_ref[...], w_ref[...], preferred_element_type=jnp.float32)
    ys = xw * dinv
    ys0_ref[...] = ys[:, :D_HALF]
    ys1_ref[...] = ys[:, D_HALF:]


def _scale_call(x_pad, w, deg_a, deg_b):
    blk = 1024
    grid = N_PAD // blk
    return pl.pallas_call(
        _scale_body,
        grid=(grid,),
        in_specs=[
            pl.BlockSpec((blk, D_FEAT), lambda i: (i, 0)),
            pl.BlockSpec((D_FEAT, D_FEAT), lambda i: (0, 0)),
            pl.BlockSpec((blk, 16), lambda i: (i, 0)),
            pl.BlockSpec((blk, 16), lambda i: (i, 0)),
        ],
        out_specs=[
            pl.BlockSpec((blk, D_HALF), lambda i: (i, 0)),
            pl.BlockSpec((blk, D_HALF), lambda i: (i, 0)),
        ],
        out_shape=[
            jax.ShapeDtypeStruct((N_PAD, D_HALF), jnp.float32),
            jax.ShapeDtypeStruct((N_PAD, D_HALF), jnp.float32),
        ],
    )(x_pad, w, deg_a, deg_b)


def _head_body(aa_ref, ab_ref, ys0_ref, ys1_ref, x_ref, da_ref, db_ref,
               bc_ref, w1_ref, b1_ref, w2_ref, b2_ref, w3_ref, b3_ref,
               out_ref):
    deg = 1.0 + da_ref[:, :1] + db_ref[:, :1]
    dinv = lax.rsqrt(deg)
    acc = jnp.concatenate([aa_ref[...], ab_ref[...]], axis=1)
    ys = jnp.concatenate([ys0_ref[...], ys1_ref[...]], axis=1)
    g = dinv * (acc + ys) + bc_ref[...]
    h = jnp.maximum(g, 0.0) + x_ref[...]
    h1 = jnp.maximum(
        jnp.dot(h, w1_ref[...], preferred_element_type=jnp.float32)
        + b1_ref[...], 0.0)
    h2 = jnp.maximum(
        jnp.dot(h1, w2_ref[...], preferred_element_type=jnp.float32)
        + b2_ref[...], 0.0)
    out_ref[...] = (
        jnp.dot(h2, w3_ref[...], preferred_element_type=jnp.float32)
        + b3_ref[...])


def _head_call(acc_a, acc_b, ys0, ys1, x_pad, deg_a, deg_b,
               bc, w1, b1, w2, b2, w3, b3):
    blk = 1024
    grid = N_PAD // blk
    full = lambda r, c_: pl.BlockSpec((r, c_), lambda i: (0, 0))
    return pl.pallas_call(
        _head_body,
        grid=(grid,),
        in_specs=[
            pl.BlockSpec((blk, D_HALF), lambda i: (i, 0)),
            pl.BlockSpec((blk, D_HALF), lambda i: (i, 0)),
            pl.BlockSpec((blk, D_HALF), lambda i: (i, 0)),
            pl.BlockSpec((blk, D_HALF), lambda i: (i, 0)),
            pl.BlockSpec((blk, D_FEAT), lambda i: (i, 0)),
            pl.BlockSpec((blk, 16), lambda i: (i, 0)),
            pl.BlockSpec((blk, 16), lambda i: (i, 0)),
            full(1, D_FEAT),
            full(D_FEAT, 32), full(1, 32),
            full(32, 32), full(1, 32),
            full(32, 1), full(1, 1),
        ],
        out_specs=pl.BlockSpec((blk, 1), lambda i: (i, 0)),
        out_shape=jax.ShapeDtypeStruct((N_PAD, 1), jnp.float32),
    )(acc_a, acc_b, ys0, ys1, x_pad, deg_a, deg_b,
      bc, w1, b1, w2, b2, w3, b3)


def kernel(x, edge_index, W_conv, b_conv, W1, b1, W2, b2, W3, b3):
    src = edge_index[0].astype(jnp.int32)
    dst = edge_index[1].astype(jnp.int32)
    n_pad_e = E_PAD - N_EDGES
    pad_idx = (N_NODES + jnp.arange(n_pad_e, dtype=jnp.int32)
               % (N_PAD - N_NODES))
    src_p = jnp.concatenate([src, pad_idx]).reshape(CHUNKS, E_CHUNK)
    dst_p = jnp.concatenate([dst, pad_idx]).reshape(CHUNKS, E_CHUNK)

    x_pad = jnp.pad(x, ((0, N_PAD - N_NODES), (0, 0)))

    deg = _deg_kernel(dst_p)
    deg_a, deg_b = deg[0], deg[1]

    ys0, ys1 = _scale_call(x_pad, W_conv, deg_a, deg_b)

    acc = _scatter_kernel(src_p, dst_p, ys0, ys1)

    out = _head_call(acc[0], acc[1], ys0, ys1, x_pad, deg_a, deg_b,
                     b_conv.reshape(1, D_FEAT),
                     W1, b1.reshape(1, 32),
                     W2, b2.reshape(1, 32),
                     W3, b3.reshape(1, 1))
    return out[:N_NODES]
